# baseline (speedup 1.0000x reference)
import functools

import jax
import jax.numpy as jnp
from jax import lax
from jax.experimental import pallas as pl
from jax.experimental.pallas import tpu as pltpu
from jax.experimental.pallas import tpu_sc as plsc

N = 4096
D = 128
NC = 2
NS = 16
NW = NC * NS
LANES = 16
BI = 256
CH = 128


def _split(X):
    Xh = X.astype(jnp.bfloat16)
    Xl = (X - Xh.astype(jnp.float32)).astype(jnp.bfloat16)
    return Xh, Xl


def _dot2(a, xh, xl):
    return (jnp.dot(a, xh, preferred_element_type=jnp.float32)
            + jnp.dot(a, xl, preferred_element_type=jnp.float32))


def _col(v, j):
    ii = lax.broadcasted_iota(jnp.int32, v.shape, 1)
    return jnp.sum(jnp.where(ii == j, v, 0.0), axis=1, keepdims=True)


def _pa_body(a_ref, xh_ref, xl_ref, y1_ref, dinv_ref):
    p = _dot2(a_ref[...], xh_ref[...], xl_ref[...])
    deg = _col(p, D)
    dinv = 1.0 / jnp.maximum(deg, 1e-8)
    y1_ref[...] = p * dinv
    dinv_ref[...] = jnp.broadcast_to(dinv, (BI, D))


def _pass_a(A_bf, X0h, X0l):
    return pl.pallas_call(
        _pa_body,
        grid=(N // BI,),
        in_specs=[
            pl.BlockSpec((BI, N), lambda i: (i, 0)),
            pl.BlockSpec((N, 2 * D), lambda i: (0, 0)),
            pl.BlockSpec((N, 2 * D), lambda i: (0, 0)),
        ],
        out_specs=[
            pl.BlockSpec((BI, 2 * D), lambda i: (i, 0)),
            pl.BlockSpec((BI, D), lambda i: (i, 0)),
        ],
        out_shape=[
            jax.ShapeDtypeStruct((N, 2 * D), jnp.float32),
            jax.ShapeDtypeStruct((N, D), jnp.float32),
        ],
    )(A_bf, X0h, X0l)


def _pb_body(a_ref, y1h_ref, y1l_ref, y1b_ref, dinv_ref, w_ref,
             h1_ref, rinv_ref):
    p = _dot2(a_ref[...], y1h_ref[...], y1l_ref[...])
    y1b = y1b_ref[...]
    dinvb = dinv_ref[...]
    dinv1 = _col(dinvb, 0)
    s = _col(y1b, D)
    ls = _col(p, D) * dinv1
    rinv = 1.0 / jnp.maximum(0.5 * s + 0.25 * ls, 1e-8)
    g = (0.5 * y1b[:, :D] + 0.25 * p[:, :D] * dinvb) * rinv
    h1_ref[...] = jnp.tanh(jnp.dot(g, w_ref[...],
                                   preferred_element_type=jnp.float32))
    rinv_ref[...] = jnp.broadcast_to(rinv, (BI, D))


def _pass_b(A_bf, Y1, Y1h, Y1l, dinvb, W0):
    return pl.pallas_call(
        _pb_body,
        grid=(N // BI,),
        in_specs=[
            pl.BlockSpec((BI, N), lambda i: (i, 0)),
            pl.BlockSpec((N, 2 * D), lambda i: (0, 0)),
            pl.BlockSpec((N, 2 * D), lambda i: (0, 0)),
            pl.BlockSpec((BI, 2 * D), lambda i: (i, 0)),
            pl.BlockSpec((BI, D), lambda i: (i, 0)),
            pl.BlockSpec((D, D), lambda i: (0, 0)),
        ],
        out_specs=[
            pl.BlockSpec((BI, D), lambda i: (i, 0)),
            pl.BlockSpec((BI, D), lambda i: (i, 0)),
        ],
        out_shape=[
            jax.ShapeDtypeStruct((N, D), jnp.float32),
            jax.ShapeDtypeStruct((N, D), jnp.float32),
        ],
    )(A_bf, Y1h, Y1l, Y1, dinvb, W0)


def _pc_body(a_ref, xh_ref, xl_ref, dinv_ref, y3_ref):
    p = _dot2(a_ref[...], xh_ref[...], xl_ref[...])
    y3_ref[...] = p * dinv_ref[...]


def _pass_c(A_bf, H1h, H1l, dinvb):
    return pl.pallas_call(
        _pc_body,
        grid=(N // BI,),
        in_specs=[
            pl.BlockSpec((BI, N), lambda i: (i, 0)),
            pl.BlockSpec((N, D), lambda i: (0, 0)),
            pl.BlockSpec((N, D), lambda i: (0, 0)),
            pl.BlockSpec((BI, D), lambda i: (i, 0)),
        ],
        out_specs=pl.BlockSpec((BI, D), lambda i: (i, 0)),
        out_shape=jax.ShapeDtypeStruct((N, D), jnp.float32),
    )(A_bf, H1h, H1l, dinvb)


def _pd_body(a_ref, y3h_ref, y3l_ref, y3b_ref, dinv_ref, rinv_ref, w_ref,
             h2_ref):
    p = _dot2(a_ref[...], y3h_ref[...], y3l_ref[...])
    g = (0.5 * y3b_ref[...] + 0.25 * p * dinv_ref[...]) * rinv_ref[...]
    h2_ref[...] = jnp.tanh(jnp.dot(g, w_ref[...],
                                   preferred_element_type=jnp.float32))


def _pass_d(A_bf, Y3, Y3h, Y3l, dinvb, rinvb, W1):
    return pl.pallas_call(
        _pd_body,
        grid=(N // BI,),
        in_specs=[
            pl.BlockSpec((BI, N), lambda i: (i, 0)),
            pl.BlockSpec((N, D), lambda i: (0, 0)),
            pl.BlockSpec((N, D), lambda i: (0, 0)),
            pl.BlockSpec((BI, D), lambda i: (i, 0)),
            pl.BlockSpec((BI, D), lambda i: (i, 0)),
            pl.BlockSpec((BI, D), lambda i: (i, 0)),
            pl.BlockSpec((D, D), lambda i: (0, 0)),
        ],
        out_specs=pl.BlockSpec((BI, D), lambda i: (i, 0)),
        out_shape=jax.ShapeDtypeStruct((N, D), jnp.float32),
    )(A_bf, Y3h, Y3l, Y3, dinvb, rinvb, W1)


def _pair_loss_sc(table, sidx, didx, labels):
    B = sidx.shape[0]
    per_w = B // NW
    nch = per_w // CH
    mesh = plsc.VectorSubcoreMesh(core_axis_name="c", subcore_axis_name="s",
                                  num_cores=NC, num_subcores=NS)

    @functools.partial(
        pl.kernel,
        out_type=jax.ShapeDtypeStruct((NW, LANES), jnp.float32),
        mesh=mesh,
        scratch_types=[
            pltpu.VMEM((per_w,), jnp.int32),
            pltpu.VMEM((per_w,), jnp.int32),
            pltpu.VMEM((per_w,), jnp.float32),
            pltpu.VMEM((CH, D), jnp.float32),
            pltpu.VMEM((CH, D), jnp.float32),
            pltpu.VMEM((CH, D), jnp.float32),
            pltpu.VMEM((CH, D), jnp.float32),
            pltpu.VMEM((LANES,), jnp.float32),
            pltpu.SemaphoreType.DMA,
            pltpu.SemaphoreType.DMA,
        ],
        compiler_params=pltpu.CompilerParams(needs_layout_passes=False),
    )
    def k(table_hbm, sidx_hbm, didx_hbm, lab_hbm, out_hbm,
          sidx_v, didx_v, lab_v, srows0, drows0, srows1, drows1,
          accv, sem0, sem1):
        wid = lax.axis_index("s") * NC + lax.axis_index("c")
        base = pl.multiple_of(wid * per_w, per_w)

        pltpu.sync_copy(sidx_hbm.at[pl.ds(base, per_w)], sidx_v)
        pltpu.sync_copy(didx_hbm.at[pl.ds(base, per_w)], didx_v)
        pltpu.sync_copy(lab_hbm.at[pl.ds(base, per_w)], lab_v)

        def issue(c, srows, drows, sem):
            off = pl.multiple_of(c * CH, CH)
            pltpu.async_copy(table_hbm.at[sidx_v.at[pl.ds(off, CH)]],
                             srows, sem)
            pltpu.async_copy(table_hbm.at[didx_v.at[pl.ds(off, CH)]],
                             drows, sem)

        def drain(srows, drows, sem):
            pltpu.make_async_copy(table_hbm.at[sidx_v.at[pl.ds(0, CH)]],
                                  srows, sem).wait()
            pltpu.make_async_copy(table_hbm.at[didx_v.at[pl.ds(0, CH)]],
                                  drows, sem).wait()

        def compute(c, srows, drows, loss16):
            def group_body(g, l16):
                rowi = g * LANES + lax.iota(jnp.int32, LANES)
                lane = lax.iota(jnp.int32, LANES)
                a0 = jnp.zeros((LANES,), jnp.float32)
                a1 = jnp.zeros((LANES,), jnp.float32)
                a2 = jnp.zeros((LANES,), jnp.float32)
                a3 = jnp.zeros((LANES,), jnp.float32)
                accs = [a0, a1, a2, a3]
                for d in range(D):
                    cols = (lane + d) & (D - 1)
                    sv = plsc.load_gather(srows, [rowi, cols])
                    dv = plsc.load_gather(drows, [rowi, cols])
                    t = sv - dv
                    accs[d % 4] = accs[d % 4] + t * t
                acc = (accs[0] + accs[1]) + (accs[2] + accs[3])
                pred = jnp.exp(acc * (-1.0 / D))
                lab = plsc.load_gather(lab_v, [c * CH + rowi])
                e = lab - pred
                return l16 + e * e

            return lax.fori_loop(0, CH // LANES, group_body, loss16)

        issue(0, srows0, drows0, sem0)

        def pair_body(j, loss16):
            c0 = 2 * j
            c1 = 2 * j + 1
            issue(c1, srows1, drows1, sem1)
            drain(srows0, drows0, sem0)
            loss16 = compute(c0, srows0, drows0, loss16)

            @pl.when(c1 + 1 < nch)
            def _():
                issue(c1 + 1, srows0, drows0, sem0)

            drain(srows1, drows1, sem1)
            return compute(c1, srows1, drows1, loss16)

        loss16 = lax.fori_loop(0, nch // 2, pair_body,
                               jnp.zeros((LANES,), jnp.float32))
        accv[...] = loss16
        pltpu.sync_copy(accv, out_hbm.at[wid])

    return k(table, sidx, didx, labels)


def kernel(pairs, labels, A, embedding_states, W):
    A = A.astype(jnp.float32)
    H = embedding_states.astype(jnp.float32)
    B = pairs.shape[0]

    A_bf = A.astype(jnp.bfloat16)
    X0 = jnp.concatenate(
        [H, jnp.ones((N, 1), jnp.float32), jnp.zeros((N, D - 1), jnp.float32)],
        axis=1)
    X0h, X0l = _split(X0)
    Y1, dinvb = _pass_a(A_bf, X0h, X0l)
    Y1h, Y1l = _split(Y1)
    H1, rinvb = _pass_b(A_bf, Y1, Y1h, Y1l, dinvb, W[0])
    H1h, H1l = _split(H1)
    Y3 = _pass_c(A_bf, H1h, H1l, dinvb)
    Y3h, Y3l = _split(Y3)
    H2 = _pass_d(A_bf, Y3, Y3h, Y3l, dinvb, rinvb, W[1])

    sidx = pairs[:, 0].astype(jnp.int32)
    didx = pairs[:, 1].astype(jnp.int32)
    partials = _pair_loss_sc(H2, sidx, didx, labels.astype(jnp.float32))
    return jnp.sum(partials) / B


# speedup vs baseline: 1.0331x; 1.0331x over previous
import functools

import jax
import jax.numpy as jnp
from jax import lax
from jax.experimental import pallas as pl
from jax.experimental.pallas import tpu as pltpu
from jax.experimental.pallas import tpu_sc as plsc

N = 4096
D = 128
NC = 2
NS = 16
NW = NC * NS
LANES = 16
BI = 256
CH = 128


def _split(X):
    Xh = X.astype(jnp.bfloat16)
    Xl = (X - Xh.astype(jnp.float32)).astype(jnp.bfloat16)
    return Xh, Xl


def _dot2(a, xh, xl):
    ab = a.astype(jnp.bfloat16)
    return (jnp.dot(ab, xh, preferred_element_type=jnp.float32)
            + jnp.dot(ab, xl, preferred_element_type=jnp.float32))


def _col(v, j):
    ii = lax.broadcasted_iota(jnp.int32, v.shape, 1)
    return jnp.sum(jnp.where(ii == j, v, 0.0), axis=1, keepdims=True)


def _pa_body(a_ref, xh_ref, xl_ref, y1_ref, dinv_ref):
    p = _dot2(a_ref[...], xh_ref[...], xl_ref[...])
    deg = _col(p, D)
    dinv = 1.0 / jnp.maximum(deg, 1e-8)
    y1_ref[...] = p * dinv
    dinv_ref[...] = jnp.broadcast_to(dinv, (BI, D))


def _pass_a(A_bf, X0h, X0l):
    return pl.pallas_call(
        _pa_body,
        grid=(N // BI,),
        in_specs=[
            pl.BlockSpec((BI, N), lambda i: (i, 0)),
            pl.BlockSpec((N, 2 * D), lambda i: (0, 0)),
            pl.BlockSpec((N, 2 * D), lambda i: (0, 0)),
        ],
        out_specs=[
            pl.BlockSpec((BI, 2 * D), lambda i: (i, 0)),
            pl.BlockSpec((BI, D), lambda i: (i, 0)),
        ],
        out_shape=[
            jax.ShapeDtypeStruct((N, 2 * D), jnp.float32),
            jax.ShapeDtypeStruct((N, D), jnp.float32),
        ],
    )(A_bf, X0h, X0l)


def _pb_body(a_ref, y1h_ref, y1l_ref, y1b_ref, dinv_ref, w_ref,
             h1_ref, rinv_ref):
    p = _dot2(a_ref[...], y1h_ref[...], y1l_ref[...])
    y1b = y1b_ref[...]
    dinvb = dinv_ref[...]
    dinv1 = _col(dinvb, 0)
    s = _col(y1b, D)
    ls = _col(p, D) * dinv1
    rinv = 1.0 / jnp.maximum(0.5 * s + 0.25 * ls, 1e-8)
    g = (0.5 * y1b[:, :D] + 0.25 * p[:, :D] * dinvb) * rinv
    h1_ref[...] = jnp.tanh(jnp.dot(g, w_ref[...],
                                   preferred_element_type=jnp.float32))
    rinv_ref[...] = jnp.broadcast_to(rinv, (BI, D))


def _pass_b(A_bf, Y1, Y1h, Y1l, dinvb, W0):
    return pl.pallas_call(
        _pb_body,
        grid=(N // BI,),
        in_specs=[
            pl.BlockSpec((BI, N), lambda i: (i, 0)),
            pl.BlockSpec((N, 2 * D), lambda i: (0, 0)),
            pl.BlockSpec((N, 2 * D), lambda i: (0, 0)),
            pl.BlockSpec((BI, 2 * D), lambda i: (i, 0)),
            pl.BlockSpec((BI, D), lambda i: (i, 0)),
            pl.BlockSpec((D, D), lambda i: (0, 0)),
        ],
        out_specs=[
            pl.BlockSpec((BI, D), lambda i: (i, 0)),
            pl.BlockSpec((BI, D), lambda i: (i, 0)),
        ],
        out_shape=[
            jax.ShapeDtypeStruct((N, D), jnp.float32),
            jax.ShapeDtypeStruct((N, D), jnp.float32),
        ],
    )(A_bf, Y1h, Y1l, Y1, dinvb, W0)


def _pc_body(a_ref, xh_ref, xl_ref, dinv_ref, y3_ref):
    p = _dot2(a_ref[...], xh_ref[...], xl_ref[...])
    y3_ref[...] = p * dinv_ref[...]


def _pass_c(A_bf, H1h, H1l, dinvb):
    return pl.pallas_call(
        _pc_body,
        grid=(N // BI,),
        in_specs=[
            pl.BlockSpec((BI, N), lambda i: (i, 0)),
            pl.BlockSpec((N, D), lambda i: (0, 0)),
            pl.BlockSpec((N, D), lambda i: (0, 0)),
            pl.BlockSpec((BI, D), lambda i: (i, 0)),
        ],
        out_specs=pl.BlockSpec((BI, D), lambda i: (i, 0)),
        out_shape=jax.ShapeDtypeStruct((N, D), jnp.float32),
    )(A_bf, H1h, H1l, dinvb)


def _pd_body(a_ref, y3h_ref, y3l_ref, y3b_ref, dinv_ref, rinv_ref, w_ref,
             h2_ref):
    p = _dot2(a_ref[...], y3h_ref[...], y3l_ref[...])
    g = (0.5 * y3b_ref[...] + 0.25 * p * dinv_ref[...]) * rinv_ref[...]
    h2_ref[...] = jnp.tanh(jnp.dot(g, w_ref[...],
                                   preferred_element_type=jnp.float32))


def _pass_d(A_bf, Y3, Y3h, Y3l, dinvb, rinvb, W1):
    return pl.pallas_call(
        _pd_body,
        grid=(N // BI,),
        in_specs=[
            pl.BlockSpec((BI, N), lambda i: (i, 0)),
            pl.BlockSpec((N, D), lambda i: (0, 0)),
            pl.BlockSpec((N, D), lambda i: (0, 0)),
            pl.BlockSpec((BI, D), lambda i: (i, 0)),
            pl.BlockSpec((BI, D), lambda i: (i, 0)),
            pl.BlockSpec((BI, D), lambda i: (i, 0)),
            pl.BlockSpec((D, D), lambda i: (0, 0)),
        ],
        out_specs=pl.BlockSpec((BI, D), lambda i: (i, 0)),
        out_shape=jax.ShapeDtypeStruct((N, D), jnp.float32),
    )(A_bf, Y3h, Y3l, Y3, dinvb, rinvb, W1)


def _pair_loss_sc(table, sidx, didx, labels):
    B = sidx.shape[0]
    per_w = B // NW
    nch = per_w // CH
    mesh = plsc.VectorSubcoreMesh(core_axis_name="c", subcore_axis_name="s",
                                  num_cores=NC, num_subcores=NS)

    @functools.partial(
        pl.kernel,
        out_type=jax.ShapeDtypeStruct((NW, LANES), jnp.float32),
        mesh=mesh,
        scratch_types=[
            pltpu.VMEM((per_w,), jnp.int32),
            pltpu.VMEM((per_w,), jnp.int32),
            pltpu.VMEM((per_w,), jnp.float32),
            pltpu.VMEM((CH, D), jnp.float32),
            pltpu.VMEM((CH, D), jnp.float32),
            pltpu.VMEM((CH, D), jnp.float32),
            pltpu.VMEM((CH, D), jnp.float32),
            pltpu.VMEM((LANES,), jnp.float32),
            pltpu.SemaphoreType.DMA,
            pltpu.SemaphoreType.DMA,
        ],
        compiler_params=pltpu.CompilerParams(needs_layout_passes=False),
    )
    def k(table_hbm, sidx_hbm, didx_hbm, lab_hbm, out_hbm,
          sidx_v, didx_v, lab_v, srows0, drows0, srows1, drows1,
          accv, sem0, sem1):
        wid = lax.axis_index("s") * NC + lax.axis_index("c")
        base = pl.multiple_of(wid * per_w, per_w)

        pltpu.sync_copy(sidx_hbm.at[pl.ds(base, per_w)], sidx_v)
        pltpu.sync_copy(didx_hbm.at[pl.ds(base, per_w)], didx_v)
        pltpu.sync_copy(lab_hbm.at[pl.ds(base, per_w)], lab_v)

        def issue(c, srows, drows, sem):
            off = pl.multiple_of(c * CH, CH)
            pltpu.async_copy(table_hbm.at[sidx_v.at[pl.ds(off, CH)]],
                             srows, sem)
            pltpu.async_copy(table_hbm.at[didx_v.at[pl.ds(off, CH)]],
                             drows, sem)

        def drain(srows, drows, sem):
            pltpu.make_async_copy(table_hbm.at[sidx_v.at[pl.ds(0, CH)]],
                                  srows, sem).wait()
            pltpu.make_async_copy(table_hbm.at[didx_v.at[pl.ds(0, CH)]],
                                  drows, sem).wait()

        def compute(c, srows, drows, loss16):
            def group_body(g, l16):
                rowi = g * LANES + lax.iota(jnp.int32, LANES)
                lane = lax.iota(jnp.int32, LANES)
                a0 = jnp.zeros((LANES,), jnp.float32)
                a1 = jnp.zeros((LANES,), jnp.float32)
                a2 = jnp.zeros((LANES,), jnp.float32)
                a3 = jnp.zeros((LANES,), jnp.float32)
                accs = [a0, a1, a2, a3]
                for d in range(D):
                    cols = (lane + d) & (D - 1)
                    sv = plsc.load_gather(srows, [rowi, cols])
                    dv = plsc.load_gather(drows, [rowi, cols])
                    t = sv - dv
                    accs[d % 4] = accs[d % 4] + t * t
                acc = (accs[0] + accs[1]) + (accs[2] + accs[3])
                pred = jnp.exp(acc * (-1.0 / D))
                lab = plsc.load_gather(lab_v, [c * CH + rowi])
                e = lab - pred
                return l16 + e * e

            return lax.fori_loop(0, CH // LANES, group_body, loss16)

        issue(0, srows0, drows0, sem0)

        def pair_body(j, loss16):
            c0 = 2 * j
            c1 = 2 * j + 1
            issue(c1, srows1, drows1, sem1)
            drain(srows0, drows0, sem0)
            loss16 = compute(c0, srows0, drows0, loss16)

            @pl.when(c1 + 1 < nch)
            def _():
                issue(c1 + 1, srows0, drows0, sem0)

            drain(srows1, drows1, sem1)
            return compute(c1, srows1, drows1, loss16)

        loss16 = lax.fori_loop(0, nch // 2, pair_body,
                               jnp.zeros((LANES,), jnp.float32))
        accv[...] = loss16
        pltpu.sync_copy(accv, out_hbm.at[wid])

    return k(table, sidx, didx, labels)


def kernel(pairs, labels, A, embedding_states, W):
    A = A.astype(jnp.float32)
    H = embedding_states.astype(jnp.float32)
    B = pairs.shape[0]

    A_bf = A
    X0 = jnp.concatenate(
        [H, jnp.ones((N, 1), jnp.float32), jnp.zeros((N, D - 1), jnp.float32)],
        axis=1)
    X0h, X0l = _split(X0)
    Y1, dinvb = _pass_a(A_bf, X0h, X0l)
    Y1h, Y1l = _split(Y1)
    H1, rinvb = _pass_b(A_bf, Y1, Y1h, Y1l, dinvb, W[0])
    H1h, H1l = _split(H1)
    Y3 = _pass_c(A_bf, H1h, H1l, dinvb)
    Y3h, Y3l = _split(Y3)
    H2 = _pass_d(A_bf, Y3, Y3h, Y3l, dinvb, rinvb, W[1])

    sidx = pairs[:, 0].astype(jnp.int32)
    didx = pairs[:, 1].astype(jnp.int32)
    partials = _pair_loss_sc(H2, sidx, didx, labels.astype(jnp.float32))
    return jnp.sum(partials) / B


# speedup vs baseline: 1.1403x; 1.1038x over previous
import functools

import jax
import jax.numpy as jnp
from jax import lax
from jax.experimental import pallas as pl
from jax.experimental.pallas import tpu as pltpu
from jax.experimental.pallas import tpu_sc as plsc

N = 4096
D = 128
NC = 2
NS = 16
NW = NC * NS
LANES = 16
BI = 256
CH = 128


def _col(v, j):
    ii = lax.broadcasted_iota(jnp.int32, v.shape, 1)
    return jnp.sum(jnp.where(ii == j, v, 0.0), axis=1, keepdims=True)


NB = N // BI


def _tc_body(a_hbm, x0_ref, w0_ref, w1_ref, h2_ref,
             abf_v, blk0, blk1, xh2_v, xl2_v, y1_v, dinv_v, rinv_v,
             h1_v, xh1_v, xl1_v, y3_v, sem0, sem1):
    blks = (blk0, blk1)
    sems = (sem0, sem1)

    def split_to(src_read, xh_v, xl_v):
        for i in range(NB):
            x = src_read(i)
            xh = x.astype(jnp.bfloat16)
            xh_v[pl.ds(i * BI, BI), :] = xh
            xl_v[pl.ds(i * BI, BI), :] = (
                x - xh.astype(jnp.float32)).astype(jnp.bfloat16)

    def dot2(ab, xh_v, xl_v):
        return (jnp.dot(ab, xh_v[...], preferred_element_type=jnp.float32)
                + jnp.dot(ab, xl_v[...], preferred_element_type=jnp.float32))

    pltpu.make_async_copy(a_hbm.at[pl.ds(0, BI), :], blk0, sem0).start()
    split_to(lambda i: x0_ref[pl.ds(i * BI, BI), :], xh2_v, xl2_v)

    for i in range(NB):
        pltpu.make_async_copy(a_hbm.at[pl.ds(i * BI, BI), :],
                              blks[i % 2], sems[i % 2]).wait()
        if i + 1 < NB:
            pltpu.make_async_copy(a_hbm.at[pl.ds((i + 1) * BI, BI), :],
                                  blks[(i + 1) % 2], sems[(i + 1) % 2]).start()
        ab = blks[i % 2][...].astype(jnp.bfloat16)
        abf_v[pl.ds(i * BI, BI), :] = ab
        p = dot2(ab, xh2_v, xl2_v)
        deg = _col(p, D)
        dinv = 1.0 / jnp.maximum(deg, 1e-8)
        y1_v[pl.ds(i * BI, BI), :] = p * dinv
        dinv_v[pl.ds(i * BI, BI), :] = jnp.broadcast_to(dinv, (BI, D))

    split_to(lambda i: y1_v[pl.ds(i * BI, BI), :], xh2_v, xl2_v)
    for i in range(NB):
        ab = abf_v[pl.ds(i * BI, BI), :]
        p = dot2(ab, xh2_v, xl2_v)
        y1b = y1_v[pl.ds(i * BI, BI), :]
        dinvb = dinv_v[pl.ds(i * BI, BI), :]
        dinv1 = _col(dinvb, 0)
        s = _col(y1b, D)
        ls = _col(p, D) * dinv1
        rinv = 1.0 / jnp.maximum(0.5 * s + 0.25 * ls, 1e-8)
        g = (0.5 * y1b[:, :D] + 0.25 * p[:, :D] * dinvb) * rinv
        h1_v[pl.ds(i * BI, BI), :] = jnp.tanh(
            jnp.dot(g, w0_ref[...], preferred_element_type=jnp.float32))
        rinv_v[pl.ds(i * BI, BI), :] = jnp.broadcast_to(rinv, (BI, D))

    split_to(lambda i: h1_v[pl.ds(i * BI, BI), :], xh1_v, xl1_v)
    for i in range(NB):
        ab = abf_v[pl.ds(i * BI, BI), :]
        p = dot2(ab, xh1_v, xl1_v)
        y3_v[pl.ds(i * BI, BI), :] = p * dinv_v[pl.ds(i * BI, BI), :]

    split_to(lambda i: y3_v[pl.ds(i * BI, BI), :], xh1_v, xl1_v)
    for i in range(NB):
        ab = abf_v[pl.ds(i * BI, BI), :]
        p = dot2(ab, xh1_v, xl1_v)
        g = (0.5 * y3_v[pl.ds(i * BI, BI), :]
             + 0.25 * p * dinv_v[pl.ds(i * BI, BI), :]) \
            * rinv_v[pl.ds(i * BI, BI), :]
        h2_ref[pl.ds(i * BI, BI), :] = jnp.tanh(
            jnp.dot(g, w1_ref[...], preferred_element_type=jnp.float32))


def _propagate(A, X0, W0, W1):
    return pl.pallas_call(
        _tc_body,
        in_specs=[
            pl.BlockSpec(memory_space=pltpu.MemorySpace.HBM),
            pl.BlockSpec(memory_space=pltpu.MemorySpace.VMEM),
            pl.BlockSpec(memory_space=pltpu.MemorySpace.VMEM),
            pl.BlockSpec(memory_space=pltpu.MemorySpace.VMEM),
        ],
        out_specs=pl.BlockSpec(memory_space=pltpu.MemorySpace.VMEM),
        out_shape=jax.ShapeDtypeStruct((N, D), jnp.float32),
        scratch_shapes=[
            pltpu.VMEM((N, N), jnp.bfloat16),
            pltpu.VMEM((BI, N), jnp.float32),
            pltpu.VMEM((BI, N), jnp.float32),
            pltpu.VMEM((N, 2 * D), jnp.bfloat16),
            pltpu.VMEM((N, 2 * D), jnp.bfloat16),
            pltpu.VMEM((N, 2 * D), jnp.float32),
            pltpu.VMEM((N, D), jnp.float32),
            pltpu.VMEM((N, D), jnp.float32),
            pltpu.VMEM((N, D), jnp.float32),
            pltpu.VMEM((N, D), jnp.bfloat16),
            pltpu.VMEM((N, D), jnp.bfloat16),
            pltpu.VMEM((N, D), jnp.float32),
            pltpu.SemaphoreType.DMA,
            pltpu.SemaphoreType.DMA,
        ],
        compiler_params=pltpu.CompilerParams(
            vmem_limit_bytes=110 * 1024 * 1024),
    )(A, X0, W0, W1)


def _pair_loss_sc(table, sidx, didx, labels):
    B = sidx.shape[0]
    per_w = B // NW
    nch = per_w // CH
    mesh = plsc.VectorSubcoreMesh(core_axis_name="c", subcore_axis_name="s",
                                  num_cores=NC, num_subcores=NS)

    @functools.partial(
        pl.kernel,
        out_type=jax.ShapeDtypeStruct((NW, LANES), jnp.float32),
        mesh=mesh,
        scratch_types=[
            pltpu.VMEM((per_w,), jnp.int32),
            pltpu.VMEM((per_w,), jnp.int32),
            pltpu.VMEM((per_w,), jnp.float32),
            pltpu.VMEM((CH, D), jnp.float32),
            pltpu.VMEM((CH, D), jnp.float32),
            pltpu.VMEM((CH, D), jnp.float32),
            pltpu.VMEM((CH, D), jnp.float32),
            pltpu.VMEM((LANES,), jnp.float32),
            pltpu.SemaphoreType.DMA,
            pltpu.SemaphoreType.DMA,
        ],
        compiler_params=pltpu.CompilerParams(needs_layout_passes=False),
    )
    def k(table_hbm, sidx_hbm, didx_hbm, lab_hbm, out_hbm,
          sidx_v, didx_v, lab_v, srows0, drows0, srows1, drows1,
          accv, sem0, sem1):
        wid = lax.axis_index("s") * NC + lax.axis_index("c")
        base = pl.multiple_of(wid * per_w, per_w)

        pltpu.sync_copy(sidx_hbm.at[pl.ds(base, per_w)], sidx_v)
        pltpu.sync_copy(didx_hbm.at[pl.ds(base, per_w)], didx_v)
        pltpu.sync_copy(lab_hbm.at[pl.ds(base, per_w)], lab_v)

        def issue(c, srows, drows, sem):
            off = pl.multiple_of(c * CH, CH)
            pltpu.async_copy(table_hbm.at[sidx_v.at[pl.ds(off, CH)]],
                             srows, sem)
            pltpu.async_copy(table_hbm.at[didx_v.at[pl.ds(off, CH)]],
                             drows, sem)

        def drain(srows, drows, sem):
            pltpu.make_async_copy(table_hbm.at[sidx_v.at[pl.ds(0, CH)]],
                                  srows, sem).wait()
            pltpu.make_async_copy(table_hbm.at[didx_v.at[pl.ds(0, CH)]],
                                  drows, sem).wait()

        def compute(c, srows, drows, loss16):
            def group_body(g, l16):
                rowi = g * LANES + lax.iota(jnp.int32, LANES)
                lane = lax.iota(jnp.int32, LANES)
                a0 = jnp.zeros((LANES,), jnp.float32)
                a1 = jnp.zeros((LANES,), jnp.float32)
                a2 = jnp.zeros((LANES,), jnp.float32)
                a3 = jnp.zeros((LANES,), jnp.float32)
                accs = [a0, a1, a2, a3]
                for d in range(D):
                    cols = (lane + d) & (D - 1)
                    sv = plsc.load_gather(srows, [rowi, cols])
                    dv = plsc.load_gather(drows, [rowi, cols])
                    t = sv - dv
                    accs[d % 4] = accs[d % 4] + t * t
                acc = (accs[0] + accs[1]) + (accs[2] + accs[3])
                pred = jnp.exp(acc * (-1.0 / D))
                lab = plsc.load_gather(lab_v, [c * CH + rowi])
                e = lab - pred
                return l16 + e * e

            return lax.fori_loop(0, CH // LANES, group_body, loss16)

        issue(0, srows0, drows0, sem0)

        def pair_body(j, loss16):
            c0 = 2 * j
            c1 = 2 * j + 1
            issue(c1, srows1, drows1, sem1)
            drain(srows0, drows0, sem0)
            loss16 = compute(c0, srows0, drows0, loss16)

            @pl.when(c1 + 1 < nch)
            def _():
                issue(c1 + 1, srows0, drows0, sem0)

            drain(srows1, drows1, sem1)
            return compute(c1, srows1, drows1, loss16)

        loss16 = lax.fori_loop(0, nch // 2, pair_body,
                               jnp.zeros((LANES,), jnp.float32))
        accv[...] = loss16
        pltpu.sync_copy(accv, out_hbm.at[wid])

    return k(table, sidx, didx, labels)


def kernel(pairs, labels, A, embedding_states, W):
    A = A.astype(jnp.float32)
    H = embedding_states.astype(jnp.float32)
    B = pairs.shape[0]

    X0 = jnp.concatenate(
        [H, jnp.ones((N, 1), jnp.float32), jnp.zeros((N, D - 1), jnp.float32)],
        axis=1)
    H2 = _propagate(A, X0, W[0], W[1])

    sidx = pairs[:, 0].astype(jnp.int32)
    didx = pairs[:, 1].astype(jnp.int32)
    partials = _pair_loss_sc(H2, sidx, didx, labels.astype(jnp.float32))
    return jnp.sum(partials) / B
